# baseline (speedup 1.0000x reference)
import functools

import jax
import jax.numpy as jnp
from jax import lax
from jax.experimental import pallas as pl
from jax.experimental.pallas import tpu as pltpu
from jax.experimental.pallas import tpu_sc as plsc

_LABEL_SMOOTHING = 0.1
_CONFIDENCE = 1.0 - _LABEL_SMOOTHING
_RB = 32
_NC = 2
_NSUB = 16
_NW = _NC * _NSUB
_B_SC = 512
_CHUNK = 2048


def _xlogy(x):
    safe = jnp.where(x > 0, x, 1.0)
    return jnp.where(x > 0, x * jnp.log(safe), 0.0)


def _xlogy_const(x):
    import math
    return x * math.log(x) if x > 0 else 0.0


def _dense_body(h_ref, out_ref, res_ref, *, b):
    j = pl.program_id(0)
    blk = out_ref[...]
    h = h_ref[...]
    colsum = jnp.sum(blk, axis=0, keepdims=True)
    res_ref[...] = jnp.full((1, 1, 128), -jnp.sum(colsum * h),
                            dtype=jnp.float32)

    @pl.when(j == 0)
    def _():
        res_ref[...] += jnp.full((1, 1, 128), b * jnp.sum(_xlogy(h)),
                                 dtype=jnp.float32)


def _dense_partial(one_hot, output, rows):
    b, v = output.shape
    nb = rows // _RB
    return pl.pallas_call(
        functools.partial(_dense_body, b=b),
        grid=(nb,),
        in_specs=[
            pl.BlockSpec((1, v), lambda j: (0, 0)),
            pl.BlockSpec((_RB, v), lambda j: (j, 0)),
        ],
        out_specs=pl.BlockSpec((1, 1, 128), lambda j: (j, 0, 0)),
        out_shape=jax.ShapeDtypeStruct((nb, 1, 128), jnp.float32),
        compiler_params=pltpu.CompilerParams(
            dimension_semantics=("arbitrary",),
        ),
    )(one_hot, output)


def _sc_rowsum(output):
    b, v = output.shape
    rows_pw = (b - _B_SC) // _NW
    trs_pw = rows_pw // 8
    n_full = (v // 128) // (_CHUNK // 128)
    v_pad = -(-v // 128) * 128
    tail_dma = v_pad - n_full * _CHUNK
    tail_valid = v - n_full * _CHUNK
    nch = trs_pw * n_full

    mesh = plsc.VectorSubcoreMesh(core_axis_name="c", subcore_axis_name="s")

    @functools.partial(
        pl.kernel,
        out_type=jax.ShapeDtypeStruct((_NW, 16), jnp.float32),
        mesh=mesh,
        scratch_types=[
            pltpu.VMEM((8, _CHUNK), jnp.float32),
            pltpu.VMEM((8, _CHUNK), jnp.float32),
            pltpu.VMEM((8, tail_dma), jnp.float32),
            pltpu.VMEM((16,), jnp.float32),
            pltpu.SemaphoreType.DMA,
            pltpu.SemaphoreType.DMA,
            pltpu.SemaphoreType.DMA,
        ],
    )
    def rowsum_kernel(out_hbm, o_hbm, buf0, buf1, tbuf, acc_ref,
                      sem0, sem1, sem2):
        w = lax.axis_index("s") * _NC + lax.axis_index("c")
        row_base = _B_SC + w * rows_pw
        acc_ref[...] = jnp.zeros((16,), jnp.float32)

        def chunk_src(g):
            row = pl.multiple_of(row_base + (g // n_full) * 8, 8)
            col = pl.multiple_of((g % n_full) * _CHUNK, 128)
            return out_hbm.at[pl.ds(row, 8), pl.ds(col, _CHUNK)]

        def accum(bref, ncols):
            @pl.loop(0, 8)
            def _(r):
                accs = [jnp.zeros((16,), jnp.float32) for _ in range(4)]
                for c in range(ncols // 16):
                    accs[c % 4] = accs[c % 4] + bref[r, c * 16:(c + 1) * 16]
                acc_ref[...] += (accs[0] + accs[1]) + (accs[2] + accs[3])

        pltpu.async_copy(chunk_src(0), buf0, sem0)
        pltpu.async_copy(chunk_src(1), buf1, sem1)

        @pl.loop(0, nch // 2)
        def _(kk):
            g = kk * 2
            pltpu.make_async_copy(chunk_src(g), buf0, sem0).wait()
            accum(buf0, _CHUNK)

            @pl.when(g + 2 < nch)
            def _():
                pltpu.async_copy(chunk_src(g + 2), buf0, sem0)

            pltpu.make_async_copy(chunk_src(g + 1), buf1, sem1).wait()
            accum(buf1, _CHUNK)

            @pl.when(g + 3 < nch)
            def _():
                pltpu.async_copy(chunk_src(g + 3), buf1, sem1)

        for tr in range(trs_pw):
            row = pl.multiple_of(row_base + tr * 8, 8)
            col = pl.multiple_of(n_full * _CHUNK, 128)
            pltpu.async_copy(
                out_hbm.at[pl.ds(row, 8), pl.ds(col, tail_dma)],
                tbuf, sem2,
            ).wait()
            accum(tbuf, tail_valid)

        pltpu.sync_copy(acc_ref, o_hbm.at[w])

    return rowsum_kernel(output)


def _sc_gather(output, cols128):
    b, v = output.shape
    per_core = b // _NC

    mesh = plsc.ScalarSubcoreMesh(axis_name="c", num_cores=_NC)

    @functools.partial(
        pl.kernel,
        out_type=jax.ShapeDtypeStruct((8 * b, 128), jnp.float32),
        mesh=mesh,
        scratch_types=[
            pltpu.SMEM((per_core,), jnp.int32),
            pltpu.SemaphoreType.DMA,
            pltpu.SemaphoreType.DMA,
        ],
    )
    def gather_kernel(out_hbm, c128_hbm, g_hbm, tbuf, sem_t, sem_g):
        cid = lax.axis_index("c")
        base = cid * per_core
        pltpu.async_copy(
            c128_hbm.at[pl.ds(base, per_core)], tbuf, sem_t
        ).wait()

        @pl.loop(0, per_core)
        def _(i):
            c128 = pl.multiple_of(tbuf[i], 128)
            row0 = pl.multiple_of(base + (i // 8) * 8, 8)
            pltpu.async_copy(
                out_hbm.at[pl.ds(row0, 8), pl.ds(c128, 128)],
                g_hbm.at[pl.ds(pl.multiple_of((base + i) * 8, 8), 8), :],
                sem_g,
            )

        pltpu.make_async_copy(
            out_hbm.at[pl.ds(0, 8 * per_core), pl.ds(0, 128)],
            g_hbm.at[pl.ds(8 * base, 8 * per_core), :],
            sem_g,
        ).wait()

    return gather_kernel(output, cols128)


def _combine_body(p_ref, s_ref, off_ref, g_ref, res_ref, *, b, smooth):
    off = off_ref[...]
    g = g_ref[...]
    lane = jax.lax.broadcasted_iota(jnp.int32, g.shape, 1)
    gsum = jnp.sum(jnp.where(lane == off, g, 0.0))
    const = b * (_xlogy_const(_CONFIDENCE) - _xlogy_const(smooth))
    ptot = jnp.sum(p_ref[...]) / 128.0
    scsum = jnp.sum(s_ref[...])
    res_ref[0, 0] = (ptot - smooth * scsum + const
                     - (_CONFIDENCE - smooth) * gsum)


@jax.jit
def kernel(output, target, one_hot):
    b, v = output.shape
    smooth = _LABEL_SMOOTHING / (v - 2)
    tgt = target.astype(jnp.int32)
    cols128 = tgt & ~127
    i = jnp.arange(8 * b, dtype=jnp.int32)
    off = jnp.where(
        i % 8 == (i // 8) % 8, jnp.repeat(tgt & 127, 8), -1
    ).reshape(8 * b, 1)

    g = _sc_gather(output, cols128)
    ssum = _sc_rowsum(output)
    parts = _dense_partial(one_hot, output, _B_SC)
    parts = parts.reshape(parts.shape[0], 128)
    nb = parts.shape[0]

    res = pl.pallas_call(
        functools.partial(_combine_body, b=b, smooth=smooth),
        in_specs=[
            pl.BlockSpec((nb, 128), lambda: (0, 0)),
            pl.BlockSpec((_NW, 16), lambda: (0, 0)),
            pl.BlockSpec((8 * b, 1), lambda: (0, 0)),
            pl.BlockSpec((8 * b, 128), lambda: (0, 0)),
        ],
        out_specs=pl.BlockSpec(memory_space=pltpu.SMEM),
        out_shape=jax.ShapeDtypeStruct((1, 1), jnp.float32),
    )(parts, ssum, off, g)
    return res[0, 0]
